# Initial kernel scaffold; baseline (speedup 1.0000x reference)
import jax
import jax.numpy as jnp
from jax.experimental import pallas as pl


def kernel(hidden_states, W_router, gate_w, up_w, down_w):
    raise NotImplementedError("write your pallas kernel here")


# speedup vs baseline: 1.3795x; 1.3795x over previous
import jax
import jax.numpy as jnp
from jax.experimental import pallas as pl
from jax.experimental.pallas import tpu as pltpu

_TOPK = 8
_TD = 256


def _router_kernel(x_ref, w_ref, ids_ref, p_ref):
    pooled = jnp.mean(x_ref[...], axis=0, keepdims=True)
    logits = jnp.dot(pooled, w_ref[...],
                     preferred_element_type=jnp.float32)
    e = logits.shape[1]
    iota = jax.lax.broadcasted_iota(jnp.int32, (1, e), 1)
    iota_k = jax.lax.broadcasted_iota(jnp.int32, (1, _TOPK), 1)
    neg_inf = jnp.float32(-jnp.inf)

    vals = logits
    ids_acc = jnp.zeros((1, _TOPK), jnp.int32)
    sel = jnp.zeros((1, e), jnp.bool_)
    for i in range(_TOPK):
        m = jnp.max(vals)
        idx = jnp.min(jnp.where(vals == m, iota, e))
        ids_acc = jnp.where(iota_k == i, idx, ids_acc)
        sel = sel | (iota == idx)
        vals = jnp.where(iota == idx, neg_inf, vals)

    z = jnp.where(sel, logits, neg_inf)
    zmax = jnp.max(z)
    ez = jnp.where(sel, jnp.exp(z - zmax), 0.0)
    p = ez / jnp.sum(ez)
    ids_ref[...] = ids_acc
    p_ref[...] = p


def _moe_kernel(ids_ref, p_ref, x_ref, g_ref, u_ref, d_ref, o_ref):
    k = pl.program_id(0)
    dd = pl.program_id(1)
    prob = p_ref[ids_ref[k]]

    xb = x_ref[...]
    gw = g_ref[0].astype(jnp.bfloat16)
    uw = u_ref[0].astype(jnp.bfloat16)
    gate = jax.lax.dot_general(xb, gw, (((1,), (1,)), ((), ())),
                               preferred_element_type=jnp.float32)
    up = jax.lax.dot_general(xb, uw, (((1,), (1,)), ((), ())),
                             preferred_element_type=jnp.float32)
    inter = (jax.nn.silu(gate) * up) * prob
    dw = d_ref[0].astype(jnp.bfloat16)
    y = jax.lax.dot_general(inter.astype(jnp.bfloat16), dw,
                            (((1,), (1,)), ((), ())),
                            preferred_element_type=jnp.float32)

    first = (k == 0) & (dd == 0)

    @pl.when(first)
    def _():
        o_ref[...] = y

    @pl.when(jnp.logical_not(first))
    def _():
        o_ref[...] += y


def kernel(hidden_states, W_router, gate_w, up_w, down_w):
    b, s, h = hidden_states.shape
    e, dff, _ = gate_w.shape
    m = b * s
    x2d = hidden_states.reshape(m, h)

    ids2d, p2d = pl.pallas_call(
        _router_kernel,
        out_shape=(
            jax.ShapeDtypeStruct((1, _TOPK), jnp.int32),
            jax.ShapeDtypeStruct((1, e), jnp.float32),
        ),
    )(x2d, W_router)
    ids = ids2d[0]
    probs = p2d[0]

    xb = x2d.astype(jnp.bfloat16)
    n_dff = dff // _TD
    grid = (_TOPK, n_dff)
    out2d = pl.pallas_call(
        _moe_kernel,
        grid_spec=pltpu.PrefetchScalarGridSpec(
            num_scalar_prefetch=2,
            grid=grid,
            in_specs=[
                pl.BlockSpec((m, h), lambda k, d, ids, p: (0, 0)),
                pl.BlockSpec((1, _TD, h), lambda k, d, ids, p: (ids[k], d, 0)),
                pl.BlockSpec((1, _TD, h), lambda k, d, ids, p: (ids[k], d, 0)),
                pl.BlockSpec((1, h, _TD), lambda k, d, ids, p: (ids[k], 0, d)),
            ],
            out_specs=pl.BlockSpec((m, h), lambda k, d, ids, p: (0, 0)),
        ),
        out_shape=jax.ShapeDtypeStruct((m, h), jnp.float32),
        compiler_params=pltpu.CompilerParams(
            dimension_semantics=("arbitrary", "arbitrary"),
        ),
    )(ids, probs, xb, gate_w, up_w, down_w)

    return out2d.reshape(b, s, h)


# speedup vs baseline: 1.4940x; 1.0830x over previous
import jax
import jax.numpy as jnp
from jax.experimental import pallas as pl
from jax.experimental.pallas import tpu as pltpu

_TOPK = 8
_TD = 1408
_TS = 512


def _router_kernel(x_ref, w_ref, ids_ref, p_ref):
    pooled = jnp.mean(x_ref[...], axis=0, keepdims=True)
    logits = jnp.dot(pooled, w_ref[...],
                     preferred_element_type=jnp.float32)
    e = logits.shape[1]
    iota = jax.lax.broadcasted_iota(jnp.int32, (1, e), 1)
    iota_k = jax.lax.broadcasted_iota(jnp.int32, (1, _TOPK), 1)
    neg_inf = jnp.float32(-jnp.inf)

    vals = logits
    ids_acc = jnp.zeros((1, _TOPK), jnp.int32)
    sel = jnp.zeros((1, e), jnp.bool_)
    for i in range(_TOPK):
        m = jnp.max(vals)
        idx = jnp.min(jnp.where(vals == m, iota, e))
        ids_acc = jnp.where(iota_k == i, idx, ids_acc)
        sel = sel | (iota == idx)
        vals = jnp.where(iota == idx, neg_inf, vals)

    z = jnp.where(sel, logits, neg_inf)
    zmax = jnp.max(z)
    ez = jnp.where(sel, jnp.exp(z - zmax), 0.0)
    p = ez / jnp.sum(ez)
    ids_ref[...] = ids_acc
    p_ref[...] = p


def _moe_kernel(ids_ref, p_ref, x_ref, g_ref, u_ref, d_ref, o_ref):
    k = pl.program_id(0)
    dd = pl.program_id(1)
    s = pl.program_id(2)
    prob = p_ref[ids_ref[k]]

    xb = x_ref[...]
    gw = g_ref[0].astype(jnp.bfloat16)
    uw = u_ref[0].astype(jnp.bfloat16)
    gate = jax.lax.dot_general(xb, gw, (((1,), (1,)), ((), ())),
                               preferred_element_type=jnp.float32)
    up = jax.lax.dot_general(xb, uw, (((1,), (1,)), ((), ())),
                             preferred_element_type=jnp.float32)
    inter = (jax.nn.silu(gate) * up) * prob
    dw = d_ref[0].astype(jnp.bfloat16)
    y = jax.lax.dot_general(inter.astype(jnp.bfloat16), dw,
                            (((1,), (1,)), ((), ())),
                            preferred_element_type=jnp.float32)

    first = (k == 0) & (dd == 0)
    row = s * _TS

    @pl.when(first)
    def _():
        o_ref[pl.ds(row, _TS), :] = y

    @pl.when(jnp.logical_not(first))
    def _():
        o_ref[pl.ds(row, _TS), :] += y


def kernel(hidden_states, W_router, gate_w, up_w, down_w):
    b, s, h = hidden_states.shape
    e, dff, _ = gate_w.shape
    m = b * s
    x2d = hidden_states.reshape(m, h)

    ids2d, p2d = pl.pallas_call(
        _router_kernel,
        out_shape=(
            jax.ShapeDtypeStruct((1, _TOPK), jnp.int32),
            jax.ShapeDtypeStruct((1, e), jnp.float32),
        ),
    )(x2d, W_router)
    ids = ids2d[0]
    probs = p2d[0]

    xb = x2d.astype(jnp.bfloat16)
    n_dff = dff // _TD
    n_s = m // _TS
    grid = (_TOPK, n_dff, n_s)
    out2d = pl.pallas_call(
        _moe_kernel,
        grid_spec=pltpu.PrefetchScalarGridSpec(
            num_scalar_prefetch=2,
            grid=grid,
            in_specs=[
                pl.BlockSpec((_TS, h), lambda k, d, s, ids, p: (s, 0)),
                pl.BlockSpec((1, _TD, h), lambda k, d, s, ids, p: (ids[k], d, 0)),
                pl.BlockSpec((1, _TD, h), lambda k, d, s, ids, p: (ids[k], d, 0)),
                pl.BlockSpec((1, h, _TD), lambda k, d, s, ids, p: (ids[k], 0, d)),
            ],
            out_specs=pl.BlockSpec((m, h), lambda k, d, s, ids, p: (0, 0)),
        ),
        out_shape=jax.ShapeDtypeStruct((m, h), jnp.float32),
        compiler_params=pltpu.CompilerParams(
            dimension_semantics=("arbitrary", "arbitrary", "arbitrary"),
        ),
    )(ids, probs, xb, gate_w, up_w, down_w)

    return out2d.reshape(b, s, h)
